# baseline (speedup 1.0000x reference)
import functools

import jax
import jax.numpy as jnp
from jax.experimental import pallas as pl

BM = 256
N = 4096


def _dot(a, b):
    return jnp.dot(a, b, preferred_element_type=jnp.float32)


def _proj_body(u_attr, v_attr, Wn_u, Ws_u, b_u, Wn_v, Ws_v, b_v,
               P_u, S_u, P_v, S_v):
    P_u[...] = _dot(v_attr[...], Wn_u[...])
    S_u[...] = _dot(u_attr[...], Ws_u[...]) + b_u[...]
    P_v[...] = _dot(u_attr[...], Wn_v[...])
    S_v[...] = _dot(v_attr[...], Ws_v[...]) + b_v[...]


def _u1_body(adj, P_u, S_u, W_in1, Q1):
    u1 = jnp.maximum(_dot(adj[...], P_u[...]) + S_u[...], 0.0)
    Q1[...] = _dot(u1, W_in1[...])


def _in1_body(adj, Q1, b_in1, Wn_v2, Ws_u2, b_u2, Pv2, Su2):
    t = jnp.maximum(_dot(adj[...], Q1[...]) + b_in1[...], 0.0)
    Pv2[...] = _dot(t, Wn_v2[...])
    Su2[...] = _dot(t, Ws_u2[...]) + b_u2[...]


def _v_body(adj, P_v, S_v, Pv2, Ws_v2, b_v2, Wn_u2, v_emb2, Pu2):
    a = adj[...]
    v_emb = jnp.maximum(_dot(a, P_v[...]) + S_v[...], 0.0)
    v_emb2[...] = jnp.maximum(
        _dot(a, Pv2[...]) + _dot(v_emb, Ws_v2[...]) + b_v2[...], 0.0)
    Pu2[...] = _dot(v_emb, Wn_u2[...])


def _u2_body(adj, Pu2, Su2, W_in2, Q2):
    u2a = jnp.maximum(_dot(adj[...], Pu2[...]) + Su2[...], 0.0)
    Q2[...] = _dot(u2a, W_in2[...])


def _fin_body(adj, Q2, b_in2, v_emb2T, u_emb2, rating):
    t = jnp.maximum(_dot(adj[...], Q2[...]) + b_in2[...], 0.0)
    u_emb2[...] = t
    rating[...] = jnp.tanh(_dot(t, v_emb2T[...]))


def _full(shape):
    return pl.BlockSpec(shape, lambda i: (0,) * len(shape))


def _rows(w):
    return pl.BlockSpec((BM, w), lambda i: (i, 0))


@jax.jit
def kernel(u_attr, v_attr, u_adj, v_adj, u_adj_inner,
           Wn_v, Ws_v, b_v, Wn_u, Ws_u, b_u, W_in1, b_in1,
           Wn_v2, Ws_v2, b_v2, Wn_u2, Ws_u2, b_u2, W_in2, b_in2):
    f32 = jnp.float32
    H = Wn_v.shape[1]
    O = Wn_v2.shape[1]
    b_v = b_v.reshape(1, H)
    b_u = b_u.reshape(1, H)
    b_in1 = b_in1.reshape(1, H)
    b_v2 = b_v2.reshape(1, O)
    b_u2 = b_u2.reshape(1, O)
    b_in2 = b_in2.reshape(1, O)
    grid = (N // BM,)

    P_u, S_u, P_v, S_v = pl.pallas_call(
        _proj_body,
        out_shape=[jax.ShapeDtypeStruct((N, H), f32)] * 4,
    )(u_attr, v_attr, Wn_u, Ws_u, b_u, Wn_v, Ws_v, b_v)

    Q1 = pl.pallas_call(
        _u1_body,
        grid=grid,
        in_specs=[_rows(N), _full((N, H)), _rows(H), _full((H, H))],
        out_specs=_rows(H),
        out_shape=jax.ShapeDtypeStruct((N, H), f32),
    )(u_adj, P_u, S_u, W_in1)

    Pv2, Su2 = pl.pallas_call(
        _in1_body,
        grid=grid,
        in_specs=[_rows(N), _full((N, H)), _full((1, H)),
                  _full((H, O)), _full((H, O)), _full((1, O))],
        out_specs=[_rows(O), _rows(O)],
        out_shape=[jax.ShapeDtypeStruct((N, O), f32)] * 2,
    )(u_adj_inner, Q1, b_in1, Wn_v2, Ws_u2, b_u2)

    v_emb2, Pu2 = pl.pallas_call(
        _v_body,
        grid=grid,
        in_specs=[_rows(N), _full((N, H)), _rows(H), _full((N, O)),
                  _full((H, O)), _full((1, O)), _full((H, O))],
        out_specs=[_rows(O), _rows(O)],
        out_shape=[jax.ShapeDtypeStruct((N, O), f32)] * 2,
    )(v_adj, P_v, S_v, Pv2, Ws_v2, b_v2, Wn_u2)

    Q2 = pl.pallas_call(
        _u2_body,
        grid=grid,
        in_specs=[_rows(N), _full((N, O)), _rows(O), _full((O, O))],
        out_specs=_rows(O),
        out_shape=jax.ShapeDtypeStruct((N, O), f32),
    )(u_adj, Pu2, Su2, W_in2)

    u_emb2, rating = pl.pallas_call(
        _fin_body,
        grid=grid,
        in_specs=[_rows(N), _full((N, O)), _full((1, O)), _full((O, N))],
        out_specs=[_rows(O), _rows(N)],
        out_shape=[jax.ShapeDtypeStruct((N, O), f32),
                   jax.ShapeDtypeStruct((N, N), f32)],
    )(u_adj_inner, Q2, b_in2, v_emb2.T)

    return (u_emb2, v_emb2, rating)


# speedup vs baseline: 1.1837x; 1.1837x over previous
import jax
import jax.numpy as jnp
from jax.experimental import pallas as pl
from jax.experimental.pallas import tpu as pltpu

BM = 256
N = 4096
NB = N // BM


def _dot(a, b):
    return jnp.dot(a, b, preferred_element_type=jnp.float32)


def _bdot(a, b):
    return jnp.dot(a.astype(jnp.bfloat16), b.astype(jnp.bfloat16),
                   preferred_element_type=jnp.float32)


def _body(u_adj, inner, v_adj, u_attr, v_attr,
          Wn_v, Ws_v, b_v, Wn_u, Ws_u, b_u, W_in1, b_in1,
          Wn_v2, Ws_v2, b_v2, Wn_u2, Ws_u2, b_u2, W_in2, b_in2,
          u_emb2, v_emb2, rating,
          P_u, S_u, P_v, S_v, Q1, Pv2, Su2, Pu2, Q2, VT):
    i = pl.program_id(0)
    p = i // NB
    r = i % NB
    row = pl.ds(r * BM, BM)

    @pl.when(i == 0)
    def _():
        P_u[...] = _dot(v_attr[...], Wn_u[...])
        S_u[...] = _dot(u_attr[...], Ws_u[...]) + b_u[...]
        P_v[...] = _dot(u_attr[...], Wn_v[...])
        S_v[...] = _dot(v_attr[...], Ws_v[...]) + b_v[...]

    @pl.when(p == 0)
    def _():
        u1 = jnp.maximum(_bdot(u_adj[...], P_u[...]) + S_u[row, :], 0.0)
        Q1[row, :] = _dot(u1, W_in1[...])

    @pl.when(p == 1)
    def _():
        t = jnp.maximum(_bdot(inner[...], Q1[...]) + b_in1[...], 0.0)
        Pv2[row, :] = _dot(t, Wn_v2[...])
        Su2[row, :] = _dot(t, Ws_u2[...]) + b_u2[...]

    @pl.when(p == 2)
    def _():
        a = v_adj[...]
        ve = jnp.maximum(_bdot(a, P_v[...]) + S_v[row, :], 0.0)
        ve2 = jnp.maximum(
            _bdot(a, Pv2[...]) + _dot(ve, Ws_v2[...]) + b_v2[...], 0.0)
        v_emb2[...] = ve2
        VT[:, row] = ve2.T
        Pu2[row, :] = _dot(ve, Wn_u2[...])

    @pl.when(p == 3)
    def _():
        u2a = jnp.maximum(_bdot(u_adj[...], Pu2[...]) + Su2[row, :], 0.0)
        Q2[row, :] = _dot(u2a, W_in2[...])

    @pl.when(p == 4)
    def _():
        t = jnp.maximum(_bdot(inner[...], Q2[...]) + b_in2[...], 0.0)
        u_emb2[...] = t
        rating[...] = jnp.tanh(_bdot(t, VT[...]))


def _im_uadj(i):
    r = i % NB
    return (jnp.where(i < NB, r, jnp.where(i < 3 * NB, 0,
            jnp.where(i < 4 * NB, r, NB - 1))), 0)


def _im_inner(i):
    r = i % NB
    return (jnp.where(i < NB, 0, jnp.where(i < 2 * NB, r,
            jnp.where(i < 4 * NB, 0, r))), 0)


def _im_vadj(i):
    r = i % NB
    return (jnp.where(i < 2 * NB, 0, jnp.where(i < 3 * NB, r, NB - 1)), 0)


def _im_p2out(i):
    r = i % NB
    return (jnp.where(i < 2 * NB, 0, jnp.where(i < 3 * NB, r, NB - 1)), 0)


def _im_p4out(i):
    r = i % NB
    return (jnp.where(i < 4 * NB, 0, r), 0)


def _full(shape):
    return pl.BlockSpec(shape, lambda i: (0,) * len(shape))


@jax.jit
def kernel(u_attr, v_attr, u_adj, v_adj, u_adj_inner,
           Wn_v, Ws_v, b_v, Wn_u, Ws_u, b_u, W_in1, b_in1,
           Wn_v2, Ws_v2, b_v2, Wn_u2, Ws_u2, b_u2, W_in2, b_in2):
    f32 = jnp.float32
    H = Wn_v.shape[1]
    O = Wn_v2.shape[1]
    DU = u_attr.shape[1]
    DV = v_attr.shape[1]
    b_v = b_v.reshape(1, H)
    b_u = b_u.reshape(1, H)
    b_in1 = b_in1.reshape(1, H)
    b_v2 = b_v2.reshape(1, O)
    b_u2 = b_u2.reshape(1, O)
    b_in2 = b_in2.reshape(1, O)

    u_emb2, v_emb2, rating = pl.pallas_call(
        _body,
        grid=(5 * NB,),
        in_specs=[
            pl.BlockSpec((BM, N), _im_uadj),
            pl.BlockSpec((BM, N), _im_inner),
            pl.BlockSpec((BM, N), _im_vadj),
            _full((N, DU)), _full((N, DV)),
            _full((DU, H)), _full((DV, H)), _full((1, H)),
            _full((DV, H)), _full((DU, H)), _full((1, H)),
            _full((H, H)), _full((1, H)),
            _full((H, O)), _full((H, O)), _full((1, O)),
            _full((H, O)), _full((H, O)), _full((1, O)),
            _full((O, O)), _full((1, O)),
        ],
        out_specs=[
            pl.BlockSpec((BM, O), _im_p4out),
            pl.BlockSpec((BM, O), _im_p2out),
            pl.BlockSpec((BM, N), _im_p4out),
        ],
        out_shape=[
            jax.ShapeDtypeStruct((N, O), f32),
            jax.ShapeDtypeStruct((N, O), f32),
            jax.ShapeDtypeStruct((N, N), f32),
        ],
        scratch_shapes=[
            pltpu.VMEM((N, H), f32), pltpu.VMEM((N, H), f32),
            pltpu.VMEM((N, H), f32), pltpu.VMEM((N, H), f32),
            pltpu.VMEM((N, H), f32),
            pltpu.VMEM((N, O), f32), pltpu.VMEM((N, O), f32),
            pltpu.VMEM((N, O), f32), pltpu.VMEM((N, O), f32),
            pltpu.VMEM((O, N), f32),
        ],
    )(u_adj, u_adj_inner, v_adj, u_attr, v_attr,
      Wn_v, Ws_v, b_v, Wn_u, Ws_u, b_u, W_in1, b_in1,
      Wn_v2, Ws_v2, b_v2, Wn_u2, Ws_u2, b_u2, W_in2, b_in2)

    return (u_emb2, v_emb2, rating)
